# baseline (speedup 1.0000x reference)
import functools

import jax
import jax.numpy as jnp
from jax import lax
from jax.experimental import pallas as pl
from jax.experimental.pallas import tpu as pltpu
from jax.experimental.pallas import tpu_sc as plsc

N = 10000
D = 128
H = 8
HD = 16
NCLS = 40

NP = 10240
NC = 2
NS = 16
NW = NC * NS
CH = 112
DS = 144
DD = 16
ROWS_PER_TILE = NP // NS


def _edge_pass(src_tab, dst_tab, ids2, nch):
  assert nch % 2 == 1 and nch >= 3
  mesh = plsc.VectorSubcoreMesh(core_axis_name="c", subcore_axis_name="s")

  @functools.partial(
      pl.kernel,
      out_type=jax.ShapeDtypeStruct((NC, NP, DS), jnp.float32),
      mesh=mesh,
      compiler_params=pltpu.CompilerParams(use_tc_tiling_on_sc=False),
      scratch_types=[
          pltpu.VMEM((2, CH), jnp.int32),
          pltpu.VMEM((2, CH), jnp.int32),
          pltpu.VMEM((CH, DS), jnp.float32),
          pltpu.VMEM((CH, DS), jnp.float32),
          pltpu.VMEM((CH, DD), jnp.float32),
          pltpu.VMEM((CH, DD), jnp.float32),
          pltpu.VMEM_SHARED((NP, DS), jnp.float32),
          pltpu.SemaphoreType.DMA,
          pltpu.SemaphoreType.DMA,
          pltpu.SemaphoreType.DMA,
          pltpu.SemaphoreType.DMA,
          pltpu.SemaphoreType.DMA,
          pltpu.SemaphoreType.DMA,
      ],
  )
  def k(stab, dtab, idh, out, ibA, ibB, rowsA, rowsB, drowsA, drowsB, acc,
        siA, siB, sgA, sgB, sdA, sdB):
    c = lax.axis_index("c")
    s = lax.axis_index("s")
    wid = c * NS + s
    cbase = wid * nch

    zero = jnp.zeros((16,), jnp.float32)

    def zrow(r, carry):
      for j in range(DS // 16):
        rowsA[r, pl.ds(j * 16, 16)] = zero
      return carry

    lax.fori_loop(0, CH, zrow, 0)
    rbase = s * ROWS_PER_TILE
    for kk in range(ROWS_PER_TILE // CH):
      pltpu.sync_copy(rowsA, acc.at[pl.ds(rbase + kk * CH, CH)])
    rem = ROWS_PER_TILE % CH
    if rem:
      pltpu.sync_copy(rowsA.at[pl.ds(0, rem)],
                      acc.at[pl.ds(rbase + ROWS_PER_TILE - rem, rem)])
    plsc.subcore_barrier()

    def start_idx(i, ib, si):
      pltpu.async_copy(idh.at[cbase + i], ib, si)

    def wait_idx(i, ib, si):
      pltpu.make_async_copy(idh.at[cbase + i], ib, si).wait()

    def start_g(ib, rows, drows, sg, sd):
      pltpu.async_copy(stab.at[ib.at[0]], rows, sg)
      pltpu.async_copy(dtab.at[ib.at[1]], drows, sd)

    def wait_g(ib, rows, drows, sg, sd):
      pltpu.make_async_copy(stab.at[ib.at[0]], rows, sg).wait()
      pltpu.make_async_copy(dtab.at[ib.at[1]], drows, sd).wait()

    def compute(rows, drows):
      def edge(e, ecarry):
        av = rows[e, pl.ds(D, 16)]
        dv = drows[e, :]
        t = av + dv
        t = jnp.maximum(t, t * jnp.float32(0.2))
        ex = jnp.exp(t)
        rows[e, pl.ds(D, 16)] = ex
        for h in range(H):
          rows[e, pl.ds(h * HD, HD)] = rows[e, pl.ds(h * HD, HD)] * ex[h]
        return ecarry

      lax.fori_loop(0, CH, edge, 0)

    def scatter(rows, ib):
      pltpu.sync_copy(rows, acc.at[ib.at[1]], add=True)

    start_idx(0, ibA, siA)
    wait_idx(0, ibA, siA)
    start_g(ibA, rowsA, drowsA, sgA, sdA)
    start_idx(1, ibB, siB)

    nj = (nch - 1) // 2

    def body(j, carry):
      ka = 2 * j
      wait_idx(ka + 1, ibB, siB)
      start_g(ibB, rowsB, drowsB, sgB, sdB)
      wait_g(ibA, rowsA, drowsA, sgA, sdA)
      compute(rowsA, drowsA)
      scatter(rowsA, ibA)
      start_idx(ka + 2, ibA, siA)
      wait_idx(ka + 2, ibA, siA)
      start_g(ibA, rowsA, drowsA, sgA, sdA)
      wait_g(ibB, rowsB, drowsB, sgB, sdB)
      compute(rowsB, drowsB)
      scatter(rowsB, ibB)
      start_idx(ka + 3, ibB, siB)
      return carry

    lax.fori_loop(0, nj, body, 0)

    wait_g(ibA, rowsA, drowsA, sgA, sdA)
    compute(rowsA, drowsA)
    scatter(rowsA, ibA)
    wait_idx(nch, ibB, siB)

    plsc.subcore_barrier()
    pltpu.sync_copy(acc.at[pl.ds(s * ROWS_PER_TILE, ROWS_PER_TILE)],
                    out.at[c, pl.ds(s * ROWS_PER_TILE, ROWS_PER_TILE)])

  return k(src_tab, dst_tab, ids2)


def _tc_tables(xp, Wa, Wb):
  def body(x_ref, wa_ref, wb_ref, o1_ref, o2_ref):
    x = x_ref[...]
    o1_ref[...] = jnp.dot(x, wa_ref[...], preferred_element_type=jnp.float32)
    o2_ref[...] = jnp.dot(x, wb_ref[...], preferred_element_type=jnp.float32)

  return pl.pallas_call(
      body,
      grid=(NP // 256,),
      in_specs=[
          pl.BlockSpec((256, D), lambda i: (i, 0)),
          pl.BlockSpec((D, DS), lambda i: (0, 0)),
          pl.BlockSpec((D, DD), lambda i: (0, 0)),
      ],
      out_specs=[
          pl.BlockSpec((256, DS), lambda i: (i, 0)),
          pl.BlockSpec((256, DD), lambda i: (i, 0)),
      ],
      out_shape=[
          jax.ShapeDtypeStruct((NP, DS), jnp.float32),
          jax.ShapeDtypeStruct((NP, DD), jnp.float32),
      ],
  )(xp, Wa, Wb)


def _tc_mid(P, R, Wa, Wb):
  def body(p0_ref, p1_ref, r_ref, wa_ref, wb_ref, o1_ref, o2_ref):
    p0 = p0_ref[0]
    p1 = p1_ref[0]
    m = p0[:, :D] + p1[:, :D]
    sden = p0[:, D:D + H] + p1[:, D:D + H] + jnp.float32(1e-16)
    den = jnp.dot(sden, r_ref[...], preferred_element_type=jnp.float32)
    h = jnp.maximum(m / den, 0.0)
    o1_ref[...] = jnp.dot(h, wa_ref[...], preferred_element_type=jnp.float32)
    o2_ref[...] = jnp.dot(h, wb_ref[...], preferred_element_type=jnp.float32)

  return pl.pallas_call(
      body,
      grid=(NP // 256,),
      in_specs=[
          pl.BlockSpec((1, 256, DS), lambda i: (0, i, 0)),
          pl.BlockSpec((1, 256, DS), lambda i: (1, i, 0)),
          pl.BlockSpec((H, D), lambda i: (0, 0)),
          pl.BlockSpec((D, DS), lambda i: (0, 0)),
          pl.BlockSpec((D, DD), lambda i: (0, 0)),
      ],
      out_specs=[
          pl.BlockSpec((256, DS), lambda i: (i, 0)),
          pl.BlockSpec((256, DD), lambda i: (i, 0)),
      ],
      out_shape=[
          jax.ShapeDtypeStruct((NP, DS), jnp.float32),
          jax.ShapeDtypeStruct((NP, DD), jnp.float32),
      ],
  )(P, P, R, Wa, Wb)


def _tc_head(P, R, Wfc, bfc):
  def body(p0_ref, p1_ref, r_ref, w_ref, b_ref, o_ref):
    p0 = p0_ref[0]
    p1 = p1_ref[0]
    m = p0[:, :D] + p1[:, :D]
    sden = p0[:, D:D + H] + p1[:, D:D + H] + jnp.float32(1e-16)
    den = jnp.dot(sden, r_ref[...], preferred_element_type=jnp.float32)
    h = jnp.maximum(m / den, 0.0)
    logits = jnp.dot(h, w_ref[...], preferred_element_type=jnp.float32)
    logits = logits + b_ref[...]
    mx = jnp.max(logits, axis=1, keepdims=True)
    e = jnp.exp(logits - mx)
    lse = jnp.log(jnp.sum(e, axis=1, keepdims=True)) + mx
    o_ref[...] = logits - lse

  return pl.pallas_call(
      body,
      grid=(NP // 256,),
      in_specs=[
          pl.BlockSpec((1, 256, DS), lambda i: (0, i, 0)),
          pl.BlockSpec((1, 256, DS), lambda i: (1, i, 0)),
          pl.BlockSpec((H, D), lambda i: (0, 0)),
          pl.BlockSpec((D, NCLS), lambda i: (0, 0)),
          pl.BlockSpec((1, NCLS), lambda i: (0, 0)),
      ],
      out_specs=pl.BlockSpec((256, NCLS), lambda i: (i, 0)),
      out_shape=jax.ShapeDtypeStruct((NP, NCLS), jnp.float32),
  )(P, P, R, Wfc, bfc)


def kernel(x, edge_index, W1, a1_src, a1_dst, W2, a2_src, a2_dst, Wfc, bfc):
  E = edge_index.shape[1]
  nch = -(-E // (NW * CH))
  if nch % 2 == 0:
    nch += 1
  epad = nch * CH * NW

  src = edge_index[0].astype(jnp.int32)
  dst = edge_index[1].astype(jnp.int32)
  if epad > E:
    pad = jnp.full((epad - E,), N, jnp.int32)
    src = jnp.concatenate([src, pad])
    dst = jnp.concatenate([dst, pad])
  ids2 = jnp.stack([src.reshape(-1, CH), dst.reshape(-1, CH)], axis=1)
  ids2 = jnp.concatenate(
      [ids2, jnp.full((2, 2, CH), N, jnp.int32)], axis=0)
  xp = jnp.pad(x, ((0, NP - N), (0, 0)))

  rep = jnp.repeat(jnp.eye(H, dtype=jnp.float32), HD, axis=0)
  z8 = jnp.zeros((D, H), jnp.float32)
  As1 = a1_src.reshape(-1, 1) * rep
  Ad1 = a1_dst.reshape(-1, 1) * rep
  Wa1 = jnp.concatenate([W1, W1 @ As1, z8], axis=1)
  Wb1 = jnp.concatenate([W1 @ Ad1, z8], axis=1)
  As2 = a2_src.reshape(-1, 1) * rep
  Ad2 = a2_dst.reshape(-1, 1) * rep
  Wa2 = jnp.concatenate([W2, W2 @ As2, z8], axis=1)
  Wb2 = jnp.concatenate([W2 @ Ad2, z8], axis=1)
  R = jnp.repeat(jnp.eye(H, dtype=jnp.float32), HD, axis=1)

  t1, d1 = _tc_tables(xp, Wa1, Wb1)
  P1 = _edge_pass(t1, d1, ids2, nch)
  t2, d2 = _tc_mid(P1, R, Wa2, Wb2)
  P2 = _edge_pass(t2, d2, ids2, nch)
  out = _tc_head(P2, R, Wfc, bfc.reshape(1, NCLS))
  return out[:N]


# speedup vs baseline: 1.1570x; 1.1570x over previous
import functools

import jax
import jax.numpy as jnp
from jax import lax
from jax.experimental import pallas as pl
from jax.experimental.pallas import tpu as pltpu
from jax.experimental.pallas import tpu_sc as plsc

N = 10000
D = 128
H = 8
HD = 16
NCLS = 40

NP = 10240
NC = 2
NS = 16
NW = NC * NS
CH = 112
DS = 144
DD = 16
ROWS_PER_TILE = NP // NS


def _edge_pass(src_tab, dst_tab, ids2, nch):
  assert nch % 2 == 1 and nch >= 3
  mesh = plsc.VectorSubcoreMesh(core_axis_name="c", subcore_axis_name="s")

  @functools.partial(
      pl.kernel,
      out_type=jax.ShapeDtypeStruct((NC, NP, DS), jnp.float32),
      mesh=mesh,
      compiler_params=pltpu.CompilerParams(use_tc_tiling_on_sc=False),
      scratch_types=[
          pltpu.VMEM((2, CH), jnp.int32),
          pltpu.VMEM((2, CH), jnp.int32),
          pltpu.VMEM((CH, DS), jnp.float32),
          pltpu.VMEM((CH, DS), jnp.float32),
          pltpu.VMEM((CH, DD), jnp.float32),
          pltpu.VMEM((CH, DD), jnp.float32),
          pltpu.VMEM_SHARED((NP, DS), jnp.float32),
          pltpu.SemaphoreType.DMA,
          pltpu.SemaphoreType.DMA,
          pltpu.SemaphoreType.DMA,
          pltpu.SemaphoreType.DMA,
          pltpu.SemaphoreType.DMA,
          pltpu.SemaphoreType.DMA,
      ],
  )
  def k(stab, dtab, idh, out, ibA, ibB, rowsA, rowsB, drowsA, drowsB, acc,
        siA, siB, sgA, sgB, sdA, sdB):
    c = lax.axis_index("c")
    s = lax.axis_index("s")
    wid = c * NS + s
    cbase = wid * nch

    zero = jnp.zeros((16,), jnp.float32)

    def zrow(r, carry):
      for j in range(DS // 16):
        rowsA[r, pl.ds(j * 16, 16)] = zero
      return carry

    lax.fori_loop(0, CH, zrow, 0)
    rbase = s * ROWS_PER_TILE
    for kk in range(ROWS_PER_TILE // CH):
      pltpu.sync_copy(rowsA, acc.at[pl.ds(rbase + kk * CH, CH)])
    rem = ROWS_PER_TILE % CH
    if rem:
      pltpu.sync_copy(rowsA.at[pl.ds(0, rem)],
                      acc.at[pl.ds(rbase + ROWS_PER_TILE - rem, rem)])
    plsc.subcore_barrier()

    def start_idx(i, ib, si):
      pltpu.async_copy(idh.at[cbase + i], ib, si)

    def wait_idx(i, ib, si):
      pltpu.make_async_copy(idh.at[cbase + i], ib, si).wait()

    def start_g(ib, rows, drows, sg, sd):
      pltpu.async_copy(stab.at[ib.at[0]], rows, sg)
      pltpu.async_copy(dtab.at[ib.at[1]], drows, sd)

    def wait_g(ib, rows, drows, sg, sd):
      pltpu.make_async_copy(stab.at[ib.at[0]], rows, sg).wait()
      pltpu.make_async_copy(dtab.at[ib.at[1]], drows, sd).wait()

    def compute(rows, drows):
      def edge(e, ecarry):
        av = rows[e, pl.ds(D, 16)]
        dv = drows[e, :]
        t = av + dv
        t = jnp.maximum(t, t * jnp.float32(0.2))
        ex = jnp.exp(t)
        rows[e, pl.ds(D, 16)] = ex
        for h in range(H):
          rows[e, pl.ds(h * HD, HD)] = rows[e, pl.ds(h * HD, HD)] * ex[h]
        return ecarry

      lax.fori_loop(0, CH, edge, 0)

    def scatter(rows, ib):
      pltpu.sync_copy(rows, acc.at[ib.at[1]], add=True)

    start_idx(0, ibA, siA)
    wait_idx(0, ibA, siA)
    start_g(ibA, rowsA, drowsA, sgA, sdA)
    start_idx(1, ibB, siB)

    nj = (nch - 1) // 2

    def body(j, carry):
      ka = 2 * j
      wait_idx(ka + 1, ibB, siB)
      start_g(ibB, rowsB, drowsB, sgB, sdB)
      wait_g(ibA, rowsA, drowsA, sgA, sdA)
      compute(rowsA, drowsA)
      scatter(rowsA, ibA)
      start_idx(ka + 2, ibA, siA)
      wait_idx(ka + 2, ibA, siA)
      start_g(ibA, rowsA, drowsA, sgA, sdA)
      wait_g(ibB, rowsB, drowsB, sgB, sdB)
      compute(rowsB, drowsB)
      scatter(rowsB, ibB)

      @pl.when(j < nj - 1)
      def _():
        start_idx(ka + 3, ibB, siB)

      return carry

    lax.fori_loop(0, nj, body, 0)

    wait_g(ibA, rowsA, drowsA, sgA, sdA)
    compute(rowsA, drowsA)
    scatter(rowsA, ibA)

    plsc.subcore_barrier()
    pltpu.sync_copy(acc.at[pl.ds(s * ROWS_PER_TILE, ROWS_PER_TILE)],
                    out.at[c, pl.ds(s * ROWS_PER_TILE, ROWS_PER_TILE)])

  return k(src_tab, dst_tab, ids2)


def _tc_tables(xp, Wa, Wb):
  def body(x_ref, wa_ref, wb_ref, o1_ref, o2_ref):
    x = x_ref[...]
    o1_ref[...] = jnp.dot(x, wa_ref[...], preferred_element_type=jnp.float32)
    o2_ref[...] = jnp.dot(x, wb_ref[...], preferred_element_type=jnp.float32)

  return pl.pallas_call(
      body,
      grid=(NP // 256,),
      in_specs=[
          pl.BlockSpec((256, D), lambda i: (i, 0)),
          pl.BlockSpec((D, DS), lambda i: (0, 0)),
          pl.BlockSpec((D, DD), lambda i: (0, 0)),
      ],
      out_specs=[
          pl.BlockSpec((256, DS), lambda i: (i, 0)),
          pl.BlockSpec((256, DD), lambda i: (i, 0)),
      ],
      out_shape=[
          jax.ShapeDtypeStruct((NP, DS), jnp.float32),
          jax.ShapeDtypeStruct((NP, DD), jnp.float32),
      ],
  )(xp, Wa, Wb)


def _tc_mid(P, R, Wa, Wb):
  def body(p0_ref, p1_ref, r_ref, wa_ref, wb_ref, o1_ref, o2_ref):
    p0 = p0_ref[0]
    p1 = p1_ref[0]
    m = p0[:, :D] + p1[:, :D]
    sden = p0[:, D:D + H] + p1[:, D:D + H] + jnp.float32(1e-16)
    den = jnp.dot(sden, r_ref[...], preferred_element_type=jnp.float32)
    h = jnp.maximum(m / den, 0.0)
    o1_ref[...] = jnp.dot(h, wa_ref[...], preferred_element_type=jnp.float32)
    o2_ref[...] = jnp.dot(h, wb_ref[...], preferred_element_type=jnp.float32)

  return pl.pallas_call(
      body,
      grid=(NP // 256,),
      in_specs=[
          pl.BlockSpec((1, 256, DS), lambda i: (0, i, 0)),
          pl.BlockSpec((1, 256, DS), lambda i: (1, i, 0)),
          pl.BlockSpec((H, D), lambda i: (0, 0)),
          pl.BlockSpec((D, DS), lambda i: (0, 0)),
          pl.BlockSpec((D, DD), lambda i: (0, 0)),
      ],
      out_specs=[
          pl.BlockSpec((256, DS), lambda i: (i, 0)),
          pl.BlockSpec((256, DD), lambda i: (i, 0)),
      ],
      out_shape=[
          jax.ShapeDtypeStruct((NP, DS), jnp.float32),
          jax.ShapeDtypeStruct((NP, DD), jnp.float32),
      ],
  )(P, P, R, Wa, Wb)


def _tc_head(P, R, Wfc, bfc):
  def body(p0_ref, p1_ref, r_ref, w_ref, b_ref, o_ref):
    p0 = p0_ref[0]
    p1 = p1_ref[0]
    m = p0[:, :D] + p1[:, :D]
    sden = p0[:, D:D + H] + p1[:, D:D + H] + jnp.float32(1e-16)
    den = jnp.dot(sden, r_ref[...], preferred_element_type=jnp.float32)
    h = jnp.maximum(m / den, 0.0)
    logits = jnp.dot(h, w_ref[...], preferred_element_type=jnp.float32)
    logits = logits + b_ref[...]
    mx = jnp.max(logits, axis=1, keepdims=True)
    e = jnp.exp(logits - mx)
    lse = jnp.log(jnp.sum(e, axis=1, keepdims=True)) + mx
    o_ref[...] = logits - lse

  return pl.pallas_call(
      body,
      grid=(NP // 256,),
      in_specs=[
          pl.BlockSpec((1, 256, DS), lambda i: (0, i, 0)),
          pl.BlockSpec((1, 256, DS), lambda i: (1, i, 0)),
          pl.BlockSpec((H, D), lambda i: (0, 0)),
          pl.BlockSpec((D, NCLS), lambda i: (0, 0)),
          pl.BlockSpec((1, NCLS), lambda i: (0, 0)),
      ],
      out_specs=pl.BlockSpec((256, NCLS), lambda i: (i, 0)),
      out_shape=jax.ShapeDtypeStruct((NP, NCLS), jnp.float32),
  )(P, P, R, Wfc, bfc)


def kernel(x, edge_index, W1, a1_src, a1_dst, W2, a2_src, a2_dst, Wfc, bfc):
  E = edge_index.shape[1]
  nch = -(-E // (NW * CH))
  if nch % 2 == 0:
    nch += 1
  epad = nch * CH * NW

  src = edge_index[0].astype(jnp.int32)
  dst = edge_index[1].astype(jnp.int32)
  if epad > E:
    pad = jnp.full((epad - E,), N, jnp.int32)
    src = jnp.concatenate([src, pad])
    dst = jnp.concatenate([dst, pad])
  ids2 = jnp.stack([src.reshape(-1, CH), dst.reshape(-1, CH)], axis=1)
  xp = jnp.pad(x, ((0, NP - N), (0, 0)))

  rep = jnp.repeat(jnp.eye(H, dtype=jnp.float32), HD, axis=0)
  z8 = jnp.zeros((D, H), jnp.float32)
  As1 = a1_src.reshape(-1, 1) * rep
  Ad1 = a1_dst.reshape(-1, 1) * rep
  Wa1 = jnp.concatenate([W1, W1 @ As1, z8], axis=1)
  Wb1 = jnp.concatenate([W1 @ Ad1, z8], axis=1)
  As2 = a2_src.reshape(-1, 1) * rep
  Ad2 = a2_dst.reshape(-1, 1) * rep
  Wa2 = jnp.concatenate([W2, W2 @ As2, z8], axis=1)
  Wb2 = jnp.concatenate([W2 @ Ad2, z8], axis=1)
  R = jnp.repeat(jnp.eye(H, dtype=jnp.float32), HD, axis=1)

  t1, d1 = _tc_tables(xp, Wa1, Wb1)
  P1 = _edge_pass(t1, d1, ids2, nch)
  t2, d2 = _tc_mid(P1, R, Wa2, Wb2)
  P2 = _edge_pass(t2, d2, ids2, nch)
  out = _tc_head(P2, R, Wfc, bfc.reshape(1, NCLS))
  return out[:N]


# speedup vs baseline: 1.2183x; 1.0529x over previous
import functools

import jax
import jax.numpy as jnp
from jax import lax
from jax.experimental import pallas as pl
from jax.experimental.pallas import tpu as pltpu
from jax.experimental.pallas import tpu_sc as plsc

N = 10000
D = 128
H = 8
HD = 16
NCLS = 40

NP = 10240
NC = 2
NS = 16
NW = NC * NS
CH = 112
DS = 144
DD = 16
ROWS_PER_TILE = NP // NS


def _edge_pass(src_tab, dst_tab, ids2, nch0, nch1):
  assert nch0 % 2 == 1 and nch1 % 2 == 1 and min(nch0, nch1) >= 3
  mesh = plsc.VectorSubcoreMesh(core_axis_name="c", subcore_axis_name="s")

  @functools.partial(
      pl.kernel,
      out_type=jax.ShapeDtypeStruct((NC, NP, DS), jnp.float32),
      mesh=mesh,
      compiler_params=pltpu.CompilerParams(use_tc_tiling_on_sc=False),
      scratch_types=[
          pltpu.VMEM((2, CH), jnp.int32),
          pltpu.VMEM((2, CH), jnp.int32),
          pltpu.VMEM((CH, DS), jnp.float32),
          pltpu.VMEM((CH, DS), jnp.float32),
          pltpu.VMEM((CH, DD), jnp.float32),
          pltpu.VMEM((CH, DD), jnp.float32),
          pltpu.VMEM_SHARED((NP, DS), jnp.float32),
          pltpu.SemaphoreType.DMA,
          pltpu.SemaphoreType.DMA,
          pltpu.SemaphoreType.DMA,
          pltpu.SemaphoreType.DMA,
          pltpu.SemaphoreType.DMA,
          pltpu.SemaphoreType.DMA,
      ],
  )
  def k(stab, dtab, idh, out, ibA, ibB, rowsA, rowsB, drowsA, drowsB, acc,
        siA, siB, sgA, sgB, sdA, sdB):
    c = lax.axis_index("c")
    s = lax.axis_index("s")
    cbase = jnp.where(c == 0, s * nch0, NS * nch0 + s * nch1)
    nch = jnp.where(c == 0, nch0, nch1)

    zero = jnp.zeros((16,), jnp.float32)

    def zrow(r, carry):
      for j in range(DS // 16):
        rowsA[r, pl.ds(j * 16, 16)] = zero
      return carry

    lax.fori_loop(0, CH, zrow, 0)
    rbase = s * ROWS_PER_TILE
    for kk in range(ROWS_PER_TILE // CH):
      pltpu.sync_copy(rowsA, acc.at[pl.ds(rbase + kk * CH, CH)])
    rem = ROWS_PER_TILE % CH
    if rem:
      pltpu.sync_copy(rowsA.at[pl.ds(0, rem)],
                      acc.at[pl.ds(rbase + ROWS_PER_TILE - rem, rem)])
    plsc.subcore_barrier()

    def start_idx(i, ib, si):
      pltpu.async_copy(idh.at[cbase + i], ib, si)

    def wait_idx(i, ib, si):
      pltpu.make_async_copy(idh.at[cbase + i], ib, si).wait()

    def start_g(ib, rows, drows, sg, sd):
      pltpu.async_copy(stab.at[ib.at[0]], rows, sg)
      pltpu.async_copy(dtab.at[ib.at[1]], drows, sd)

    def wait_g(ib, rows, drows, sg, sd):
      pltpu.make_async_copy(stab.at[ib.at[0]], rows, sg).wait()
      pltpu.make_async_copy(dtab.at[ib.at[1]], drows, sd).wait()

    def compute(rows, drows):
      def edge(e, ecarry):
        av = rows[e, pl.ds(D, 16)]
        dv = drows[e, :]
        t = av + dv
        t = jnp.maximum(t, t * jnp.float32(0.2))
        ex = jnp.exp(t)
        rows[e, pl.ds(D, 16)] = ex
        for h in range(H):
          rows[e, pl.ds(h * HD, HD)] = rows[e, pl.ds(h * HD, HD)] * ex[h]
        return ecarry

      lax.fori_loop(0, CH, edge, 0)

    def scatter(rows, ib):
      pltpu.sync_copy(rows, acc.at[ib.at[1]], add=True)

    start_idx(0, ibA, siA)
    wait_idx(0, ibA, siA)
    start_g(ibA, rowsA, drowsA, sgA, sdA)
    start_idx(1, ibB, siB)

    nj = (nch - 1) // 2

    def body(j, carry):
      ka = 2 * j
      wait_idx(ka + 1, ibB, siB)
      start_g(ibB, rowsB, drowsB, sgB, sdB)
      wait_g(ibA, rowsA, drowsA, sgA, sdA)
      compute(rowsA, drowsA)
      scatter(rowsA, ibA)
      start_idx(ka + 2, ibA, siA)
      wait_idx(ka + 2, ibA, siA)
      start_g(ibA, rowsA, drowsA, sgA, sdA)
      wait_g(ibB, rowsB, drowsB, sgB, sdB)
      compute(rowsB, drowsB)
      scatter(rowsB, ibB)

      @pl.when(j < nj - 1)
      def _():
        start_idx(ka + 3, ibB, siB)

      return carry

    lax.fori_loop(0, nj, body, 0)

    wait_g(ibA, rowsA, drowsA, sgA, sdA)
    compute(rowsA, drowsA)
    scatter(rowsA, ibA)

    plsc.subcore_barrier()
    pltpu.sync_copy(acc.at[pl.ds(s * ROWS_PER_TILE, ROWS_PER_TILE)],
                    out.at[c, pl.ds(s * ROWS_PER_TILE, ROWS_PER_TILE)])

  return k(src_tab, dst_tab, ids2)


def _tc_tables(xp, Wa, Wb):
  def body(x_ref, wa_ref, wb_ref, o1_ref, o2_ref):
    x = x_ref[...]
    o1_ref[...] = jnp.dot(x, wa_ref[...], preferred_element_type=jnp.float32)
    o2_ref[...] = jnp.dot(x, wb_ref[...], preferred_element_type=jnp.float32)

  return pl.pallas_call(
      body,
      grid=(NP // 256,),
      in_specs=[
          pl.BlockSpec((256, D), lambda i: (i, 0)),
          pl.BlockSpec((D, DS), lambda i: (0, 0)),
          pl.BlockSpec((D, DD), lambda i: (0, 0)),
      ],
      out_specs=[
          pl.BlockSpec((256, DS), lambda i: (i, 0)),
          pl.BlockSpec((256, DD), lambda i: (i, 0)),
      ],
      out_shape=[
          jax.ShapeDtypeStruct((NP, DS), jnp.float32),
          jax.ShapeDtypeStruct((NP, DD), jnp.float32),
      ],
  )(xp, Wa, Wb)


def _tc_mid(P, R, Wa, Wb):
  def body(p0_ref, p1_ref, r_ref, wa_ref, wb_ref, o1_ref, o2_ref):
    p0 = p0_ref[0]
    p1 = p1_ref[0]
    m = p0[:, :D] + p1[:, :D]
    sden = p0[:, D:D + H] + p1[:, D:D + H] + jnp.float32(1e-16)
    den = jnp.dot(sden, r_ref[...], preferred_element_type=jnp.float32)
    h = jnp.maximum(m / den, 0.0)
    o1_ref[...] = jnp.dot(h, wa_ref[...], preferred_element_type=jnp.float32)
    o2_ref[...] = jnp.dot(h, wb_ref[...], preferred_element_type=jnp.float32)

  return pl.pallas_call(
      body,
      grid=(NP // 256,),
      in_specs=[
          pl.BlockSpec((1, 256, DS), lambda i: (0, i, 0)),
          pl.BlockSpec((1, 256, DS), lambda i: (1, i, 0)),
          pl.BlockSpec((H, D), lambda i: (0, 0)),
          pl.BlockSpec((D, DS), lambda i: (0, 0)),
          pl.BlockSpec((D, DD), lambda i: (0, 0)),
      ],
      out_specs=[
          pl.BlockSpec((256, DS), lambda i: (i, 0)),
          pl.BlockSpec((256, DD), lambda i: (i, 0)),
      ],
      out_shape=[
          jax.ShapeDtypeStruct((NP, DS), jnp.float32),
          jax.ShapeDtypeStruct((NP, DD), jnp.float32),
      ],
  )(P, P, R, Wa, Wb)


def _tc_head(P, R, Wfc, bfc):
  def body(p0_ref, p1_ref, r_ref, w_ref, b_ref, o_ref):
    p0 = p0_ref[0]
    p1 = p1_ref[0]
    m = p0[:, :D] + p1[:, :D]
    sden = p0[:, D:D + H] + p1[:, D:D + H] + jnp.float32(1e-16)
    den = jnp.dot(sden, r_ref[...], preferred_element_type=jnp.float32)
    h = jnp.maximum(m / den, 0.0)
    logits = jnp.dot(h, w_ref[...], preferred_element_type=jnp.float32)
    logits = logits + b_ref[...]
    mx = jnp.max(logits, axis=1, keepdims=True)
    e = jnp.exp(logits - mx)
    lse = jnp.log(jnp.sum(e, axis=1, keepdims=True)) + mx
    o_ref[...] = logits - lse

  return pl.pallas_call(
      body,
      grid=(NP // 256,),
      in_specs=[
          pl.BlockSpec((1, 256, DS), lambda i: (0, i, 0)),
          pl.BlockSpec((1, 256, DS), lambda i: (1, i, 0)),
          pl.BlockSpec((H, D), lambda i: (0, 0)),
          pl.BlockSpec((D, NCLS), lambda i: (0, 0)),
          pl.BlockSpec((1, NCLS), lambda i: (0, 0)),
      ],
      out_specs=pl.BlockSpec((256, NCLS), lambda i: (i, 0)),
      out_shape=jax.ShapeDtypeStruct((NP, NCLS), jnp.float32),
  )(P, P, R, Wfc, bfc)


def kernel(x, edge_index, W1, a1_src, a1_dst, W2, a2_src, a2_dst, Wfc, bfc):
  E = edge_index.shape[1]
  nch = -(-E // (NW * CH))
  if nch % 2 == 0:
    nch += 1
  epad = nch * CH * NW
  delta = 10
  nch0, nch1 = nch + delta, nch - delta

  src = edge_index[0].astype(jnp.int32)
  dst = edge_index[1].astype(jnp.int32)
  if epad > E:
    pad = jnp.full((epad - E,), N, jnp.int32)
    src = jnp.concatenate([src, pad])
    dst = jnp.concatenate([dst, pad])
  ids2 = jnp.stack([src.reshape(-1, CH), dst.reshape(-1, CH)], axis=1)
  xp = jnp.pad(x, ((0, NP - N), (0, 0)))

  rep = jnp.repeat(jnp.eye(H, dtype=jnp.float32), HD, axis=0)
  z8 = jnp.zeros((D, H), jnp.float32)
  As1 = a1_src.reshape(-1, 1) * rep
  Ad1 = a1_dst.reshape(-1, 1) * rep
  Wa1 = jnp.concatenate([W1, W1 @ As1, z8], axis=1)
  Wb1 = jnp.concatenate([W1 @ Ad1, z8], axis=1)
  As2 = a2_src.reshape(-1, 1) * rep
  Ad2 = a2_dst.reshape(-1, 1) * rep
  Wa2 = jnp.concatenate([W2, W2 @ As2, z8], axis=1)
  Wb2 = jnp.concatenate([W2 @ Ad2, z8], axis=1)
  R = jnp.repeat(jnp.eye(H, dtype=jnp.float32), HD, axis=1)

  t1, d1 = _tc_tables(xp, Wa1, Wb1)
  P1 = _edge_pass(t1, d1, ids2, nch0, nch1)
  t2, d2 = _tc_mid(P1, R, Wa2, Wb2)
  P2 = _edge_pass(t2, d2, ids2, nch0, nch1)
  out = _tc_head(P2, R, Wfc, bfc.reshape(1, NCLS))
  return out[:N]


# speedup vs baseline: 1.2771x; 1.0483x over previous
import functools

import jax
import jax.numpy as jnp
from jax import lax
from jax.experimental import pallas as pl
from jax.experimental.pallas import tpu as pltpu
from jax.experimental.pallas import tpu_sc as plsc

N = 10000
D = 128
H = 8
HD = 16
NCLS = 40

NP = 10240
NC = 2
NS = 16
NW = NC * NS
CH = 112
DS = 144
DD = 16
ROWS_PER_TILE = NP // NS


def _edge_pass(src_tab, dst_tab, ids2, nch0, nch1):
  assert nch0 % 2 == 1 and nch1 % 2 == 1 and min(nch0, nch1) >= 3
  mesh = plsc.VectorSubcoreMesh(core_axis_name="c", subcore_axis_name="s")

  @functools.partial(
      pl.kernel,
      out_type=jax.ShapeDtypeStruct((NC, NP, DS), jnp.float32),
      mesh=mesh,
      compiler_params=pltpu.CompilerParams(use_tc_tiling_on_sc=False),
      scratch_types=[
          pltpu.VMEM((2, CH), jnp.int32),
          pltpu.VMEM((2, CH), jnp.int32),
          pltpu.VMEM((CH, DS), jnp.float32),
          pltpu.VMEM((CH, DS), jnp.float32),
          pltpu.VMEM((CH, DD), jnp.float32),
          pltpu.VMEM((CH, DD), jnp.float32),
          pltpu.VMEM_SHARED((NP, DS), jnp.float32),
          pltpu.SemaphoreType.DMA,
          pltpu.SemaphoreType.DMA,
          pltpu.SemaphoreType.DMA,
          pltpu.SemaphoreType.DMA,
          pltpu.SemaphoreType.DMA,
          pltpu.SemaphoreType.DMA,
      ],
  )
  def k(stab, dtab, idh, out, ibA, ibB, rowsA, rowsB, drowsA, drowsB, acc,
        siA, siB, sgA, sgB, sdA, sdB):
    c = lax.axis_index("c")
    s = lax.axis_index("s")
    cbase = jnp.where(c == 0, s * nch0, NS * nch0 + s * nch1)
    nch = jnp.where(c == 0, nch0, nch1)

    zero = jnp.zeros((16,), jnp.float32)

    def zrow(r, carry):
      for j in range(DS // 16):
        rowsA[r, pl.ds(j * 16, 16)] = zero
      return carry

    lax.fori_loop(0, CH, zrow, 0)
    rbase = s * ROWS_PER_TILE
    for kk in range(ROWS_PER_TILE // CH):
      pltpu.sync_copy(rowsA, acc.at[pl.ds(rbase + kk * CH, CH)])
    rem = ROWS_PER_TILE % CH
    if rem:
      pltpu.sync_copy(rowsA.at[pl.ds(0, rem)],
                      acc.at[pl.ds(rbase + ROWS_PER_TILE - rem, rem)])
    plsc.subcore_barrier()

    def start_idx(i, ib, si):
      pltpu.async_copy(idh.at[cbase + i], ib, si)

    def wait_idx(i, ib, si):
      pltpu.make_async_copy(idh.at[cbase + i], ib, si).wait()

    def start_g(ib, rows, drows, sg, sd):
      pltpu.async_copy(stab.at[ib.at[0]], rows, sg)
      pltpu.async_copy(dtab.at[ib.at[1]], drows, sd)

    def wait_g(ib, rows, drows, sg, sd):
      pltpu.make_async_copy(stab.at[ib.at[0]], rows, sg).wait()
      pltpu.make_async_copy(dtab.at[ib.at[1]], drows, sd).wait()

    def compute(rows, drows):
      def edge(e, ecarry):
        av = rows[e, pl.ds(D, 16)]
        dv = drows[e, :]
        t = av + dv
        t = jnp.maximum(t, t * jnp.float32(0.2))
        ex = jnp.exp(t)
        rows[e, pl.ds(D, 16)] = ex
        for h in range(H):
          rows[e, pl.ds(h * HD, HD)] = rows[e, pl.ds(h * HD, HD)] * ex[h]
        return ecarry

      lax.fori_loop(0, CH, edge, 0)

    def scatter(rows, ib):
      pltpu.sync_copy(rows, acc.at[ib.at[1]], add=True)

    start_idx(0, ibA, siA)
    wait_idx(0, ibA, siA)
    start_g(ibA, rowsA, drowsA, sgA, sdA)
    start_idx(1, ibB, siB)

    nj = (nch - 1) // 2

    def body(j, carry):
      ka = 2 * j
      wait_idx(ka + 1, ibB, siB)
      start_g(ibB, rowsB, drowsB, sgB, sdB)
      wait_g(ibA, rowsA, drowsA, sgA, sdA)
      compute(rowsA, drowsA)
      scatter(rowsA, ibA)
      start_idx(ka + 2, ibA, siA)
      wait_idx(ka + 2, ibA, siA)
      start_g(ibA, rowsA, drowsA, sgA, sdA)
      wait_g(ibB, rowsB, drowsB, sgB, sdB)
      compute(rowsB, drowsB)
      scatter(rowsB, ibB)

      @pl.when(j < nj - 1)
      def _():
        start_idx(ka + 3, ibB, siB)

      return carry

    lax.fori_loop(0, nj, body, 0)

    wait_g(ibA, rowsA, drowsA, sgA, sdA)
    compute(rowsA, drowsA)
    scatter(rowsA, ibA)

    plsc.subcore_barrier()
    pltpu.sync_copy(acc.at[pl.ds(s * ROWS_PER_TILE, ROWS_PER_TILE)],
                    out.at[c, pl.ds(s * ROWS_PER_TILE, ROWS_PER_TILE)])

  return k(src_tab, dst_tab, ids2)


def _tc_tables(xp, Wa, Wb):
  def body(x_ref, wa_ref, wb_ref, o1_ref, o2_ref):
    x = x_ref[...]
    o1_ref[...] = jnp.dot(x, wa_ref[...], preferred_element_type=jnp.float32)
    o2_ref[...] = jnp.dot(x, wb_ref[...], preferred_element_type=jnp.float32)

  return pl.pallas_call(
      body,
      grid=(NP // 256,),
      in_specs=[
          pl.BlockSpec((256, D), lambda i: (i, 0)),
          pl.BlockSpec((D, DS), lambda i: (0, 0)),
          pl.BlockSpec((D, DD), lambda i: (0, 0)),
      ],
      out_specs=[
          pl.BlockSpec((256, DS), lambda i: (i, 0)),
          pl.BlockSpec((256, DD), lambda i: (i, 0)),
      ],
      out_shape=[
          jax.ShapeDtypeStruct((NP, DS), jnp.float32),
          jax.ShapeDtypeStruct((NP, DD), jnp.float32),
      ],
  )(xp, Wa, Wb)


def _tc_mid(P, R, Wa, Wb):
  def body(p0_ref, p1_ref, r_ref, wa_ref, wb_ref, o1_ref, o2_ref):
    p0 = p0_ref[0]
    p1 = p1_ref[0]
    m = p0[:, :D] + p1[:, :D]
    sden = p0[:, D:D + H] + p1[:, D:D + H] + jnp.float32(1e-16)
    den = jnp.dot(sden, r_ref[...], preferred_element_type=jnp.float32)
    h = jnp.maximum(m / den, 0.0)
    o1_ref[...] = jnp.dot(h, wa_ref[...], preferred_element_type=jnp.float32)
    o2_ref[...] = jnp.dot(h, wb_ref[...], preferred_element_type=jnp.float32)

  return pl.pallas_call(
      body,
      grid=(NP // 256,),
      in_specs=[
          pl.BlockSpec((1, 256, DS), lambda i: (0, i, 0)),
          pl.BlockSpec((1, 256, DS), lambda i: (1, i, 0)),
          pl.BlockSpec((H, D), lambda i: (0, 0)),
          pl.BlockSpec((D, DS), lambda i: (0, 0)),
          pl.BlockSpec((D, DD), lambda i: (0, 0)),
      ],
      out_specs=[
          pl.BlockSpec((256, DS), lambda i: (i, 0)),
          pl.BlockSpec((256, DD), lambda i: (i, 0)),
      ],
      out_shape=[
          jax.ShapeDtypeStruct((NP, DS), jnp.float32),
          jax.ShapeDtypeStruct((NP, DD), jnp.float32),
      ],
  )(P, P, R, Wa, Wb)


def _tc_head(P, R, Wfc, bfc):
  def body(p0_ref, p1_ref, r_ref, w_ref, b_ref, o_ref):
    p0 = p0_ref[0]
    p1 = p1_ref[0]
    m = p0[:, :D] + p1[:, :D]
    sden = p0[:, D:D + H] + p1[:, D:D + H] + jnp.float32(1e-16)
    den = jnp.dot(sden, r_ref[...], preferred_element_type=jnp.float32)
    h = jnp.maximum(m / den, 0.0)
    logits = jnp.dot(h, w_ref[...], preferred_element_type=jnp.float32)
    logits = logits + b_ref[...]
    mx = jnp.max(logits, axis=1, keepdims=True)
    e = jnp.exp(logits - mx)
    lse = jnp.log(jnp.sum(e, axis=1, keepdims=True)) + mx
    o_ref[...] = logits - lse

  return pl.pallas_call(
      body,
      grid=(NP // 256,),
      in_specs=[
          pl.BlockSpec((1, 256, DS), lambda i: (0, i, 0)),
          pl.BlockSpec((1, 256, DS), lambda i: (1, i, 0)),
          pl.BlockSpec((H, D), lambda i: (0, 0)),
          pl.BlockSpec((D, NCLS), lambda i: (0, 0)),
          pl.BlockSpec((1, NCLS), lambda i: (0, 0)),
      ],
      out_specs=pl.BlockSpec((256, NCLS), lambda i: (i, 0)),
      out_shape=jax.ShapeDtypeStruct((NP, NCLS), jnp.float32),
  )(P, P, R, Wfc, bfc)


def kernel(x, edge_index, W1, a1_src, a1_dst, W2, a2_src, a2_dst, Wfc, bfc):
  E = edge_index.shape[1]
  nch = -(-E // (NW * CH))
  if nch % 2 == 0:
    nch += 1
  epad = nch * CH * NW
  delta = 20
  nch0, nch1 = nch + delta, nch - delta

  src = edge_index[0].astype(jnp.int32)
  dst = edge_index[1].astype(jnp.int32)
  if epad > E:
    pad = jnp.full((epad - E,), N, jnp.int32)
    src = jnp.concatenate([src, pad])
    dst = jnp.concatenate([dst, pad])
  ids2 = jnp.stack([src.reshape(-1, CH), dst.reshape(-1, CH)], axis=1)
  xp = jnp.pad(x, ((0, NP - N), (0, 0)))

  rep = jnp.repeat(jnp.eye(H, dtype=jnp.float32), HD, axis=0)
  z8 = jnp.zeros((D, H), jnp.float32)
  As1 = a1_src.reshape(-1, 1) * rep
  Ad1 = a1_dst.reshape(-1, 1) * rep
  Wa1 = jnp.concatenate([W1, W1 @ As1, z8], axis=1)
  Wb1 = jnp.concatenate([W1 @ Ad1, z8], axis=1)
  As2 = a2_src.reshape(-1, 1) * rep
  Ad2 = a2_dst.reshape(-1, 1) * rep
  Wa2 = jnp.concatenate([W2, W2 @ As2, z8], axis=1)
  Wb2 = jnp.concatenate([W2 @ Ad2, z8], axis=1)
  R = jnp.repeat(jnp.eye(H, dtype=jnp.float32), HD, axis=1)

  t1, d1 = _tc_tables(xp, Wa1, Wb1)
  P1 = _edge_pass(t1, d1, ids2, nch0, nch1)
  t2, d2 = _tc_mid(P1, R, Wa2, Wb2)
  P2 = _edge_pass(t2, d2, ids2, nch0, nch1)
  out = _tc_head(P2, R, Wfc, bfc.reshape(1, NCLS))
  return out[:N]
